# baseline (speedup 1.0000x reference)
import functools
import jax
import jax.numpy as jnp
from jax import lax
from jax.experimental import pallas as pl
from jax.experimental.pallas import tpu as pltpu

NROWS = 64
NCOLS = 1_000_000
BLK = 16_384
GRID = (NCOLS + BLK - 1) // BLK
GROUPS = BLK // 128

_NOISE = None


def _gumbel_noise():
    global _NOISE
    if _NOISE is None:
        def make():
            key = jax.random.key(42)
            u = jax.random.uniform(key, (NROWS, NCOLS), dtype=jnp.float32,
                                   minval=1e-7, maxval=1.0 - 1e-7)
            return -jnp.log(-jnp.log(u))
        _NOISE = jax.jit(make)()
    return _NOISE


def _tc_body(lref, gref, outref, rmax, ridx):
    i = pl.program_id(0)

    @pl.when(i == 0)
    def _():
        rmax[...] = jnp.full((NROWS, 128), -jnp.inf, jnp.float32)
        ridx[...] = jnp.zeros((NROWS, 128), jnp.int32)

    base = i * BLK
    lane = jax.lax.broadcasted_iota(jnp.int32, (NROWS, 128), 1)

    rm = rmax[...]
    ri = ridx[...]

    def group(g, car):
        rm, ri = car
        off = g * 128
        v = lref[:, pl.ds(off, 128)] + gref[:, pl.ds(off, 128)]
        col = (base + off) + lane
        valid = col < NCOLS
        v = jnp.where(valid, v, -jnp.inf)
        m = v > rm
        rm = jnp.where(m, v, rm)
        ri = jnp.where(m, col, ri)
        return rm, ri

    rm, ri = lax.fori_loop(0, GROUPS, group, (rm, ri))
    rmax[...] = rm
    ridx[...] = ri

    @pl.when(i == GRID - 1)
    def _():
        mval = jnp.max(rm, axis=1, keepdims=True)
        cand = jnp.where(rm == mval, ri, jnp.int32(2**31 - 1))
        outref[...] = jnp.min(cand, axis=1, keepdims=True)


_tc_argmax = pl.pallas_call(
    _tc_body,
    grid=(GRID,),
    in_specs=[
        pl.BlockSpec((NROWS, BLK), lambda i: (0, i)),
        pl.BlockSpec((NROWS, BLK), lambda i: (0, i)),
    ],
    out_specs=pl.BlockSpec((NROWS, 1), lambda i: (0, 0)),
    out_shape=jax.ShapeDtypeStruct((NROWS, 1), jnp.int32),
    scratch_shapes=[
        pltpu.VMEM((NROWS, 128), jnp.float32),
        pltpu.VMEM((NROWS, 128), jnp.int32),
    ],
)


def kernel(logits):
    one = (logits[0, 0] - logits[0, 0]) + jnp.float32(1.0)
    out = _tc_argmax(logits, _gumbel_noise() * one)
    return out.reshape(NROWS)


# speedup vs baseline: 1.0099x; 1.0099x over previous
import numpy as np
import jax
import jax.numpy as jnp
from jax import lax
from jax.experimental import pallas as pl
from jax.experimental.pallas import tpu as pltpu

NROWS = 64
NCOLS = 1_000_000
BLK = 16_384
GRID = (NCOLS + BLK - 1) // BLK
GROUPS = BLK // 128

_K0 = np.uint32(0)
_K1 = np.uint32(42)
_KS2 = np.uint32((0 ^ 42) ^ 0x1BD11BDA)
_ROT_A = (13, 15, 26, 6)
_ROT_B = (17, 29, 16, 24)
_MIN = np.float32(1e-7)
_SCALE = np.float32(np.float32(1.0 - 1e-7) - np.float32(1e-7))


def _rotl(x, r):
    return (x << np.uint32(r)) | (x >> np.uint32(32 - r))


def _rounds(x0, x1, rs):
    for r in rs:
        x0 = x0 + x1
        x1 = _rotl(x1, r)
        x1 = x1 ^ x0
    return x0, x1


def _threefry_bits(flat_u32):
    x0 = jnp.full(flat_u32.shape, _K0, jnp.uint32)
    x1 = flat_u32 + _K1
    x0, x1 = _rounds(x0, x1, _ROT_A)
    x0 = x0 + _K1
    x1 = x1 + (_KS2 + np.uint32(1))
    x0, x1 = _rounds(x0, x1, _ROT_B)
    x0 = x0 + _KS2
    x1 = x1 + (_K0 + np.uint32(2))
    x0, x1 = _rounds(x0, x1, _ROT_A)
    x0 = x0 + _K0
    x1 = x1 + (_K1 + np.uint32(3))
    x0, x1 = _rounds(x0, x1, _ROT_B)
    x0 = x0 + _K1
    x1 = x1 + (_KS2 + np.uint32(4))
    x0, x1 = _rounds(x0, x1, _ROT_A)
    x0 = x0 + _KS2
    x1 = x1 + (_K0 + np.uint32(5))
    return x0 ^ x1


def _gumbel(flat_u32):
    bits = _threefry_bits(flat_u32)
    fb = (bits >> np.uint32(9)) | np.uint32(0x3F800000)
    fl = lax.bitcast_convert_type(fb, jnp.float32) - jnp.float32(1.0)
    u = jnp.maximum(_MIN, fl * _SCALE + _MIN)
    return -jnp.log(-jnp.log(u))


def _tc_body(lref, outref, rmax, ridx):
    i = pl.program_id(0)

    @pl.when(i == 0)
    def _():
        rmax[...] = jnp.full((NROWS, 128), -jnp.inf, jnp.float32)
        ridx[...] = jnp.zeros((NROWS, 128), jnp.int32)

    base = i * BLK
    lane = lax.broadcasted_iota(jnp.int32, (NROWS, 128), 1)
    rowc = lax.broadcasted_iota(jnp.int32, (NROWS, 128), 0) * NCOLS

    rm = rmax[...]
    ri = ridx[...]

    def group(g, car):
        rm, ri = car
        off = g * 128
        col = (base + off) + lane
        flat = (rowc + col).astype(jnp.uint32)
        gum = _gumbel(flat)
        v = lref[:, pl.ds(off, 128)] + gum
        valid = col < NCOLS
        v = jnp.where(valid, v, -jnp.inf)
        m = v > rm
        rm = jnp.where(m, v, rm)
        ri = jnp.where(m, col, ri)
        return rm, ri

    rm, ri = lax.fori_loop(0, GROUPS, group, (rm, ri))
    rmax[...] = rm
    ridx[...] = ri

    @pl.when(i == GRID - 1)
    def _():
        mval = jnp.max(rm, axis=1, keepdims=True)
        cand = jnp.where(rm == mval, ri, jnp.int32(2**31 - 1))
        outref[...] = jnp.min(cand, axis=1, keepdims=True)


_tc_argmax = pl.pallas_call(
    _tc_body,
    grid=(GRID,),
    in_specs=[
        pl.BlockSpec((NROWS, BLK), lambda i: (0, i)),
    ],
    out_specs=pl.BlockSpec((NROWS, 1), lambda i: (0, 0)),
    out_shape=jax.ShapeDtypeStruct((NROWS, 1), jnp.int32),
    scratch_shapes=[
        pltpu.VMEM((NROWS, 128), jnp.float32),
        pltpu.VMEM((NROWS, 128), jnp.int32),
    ],
)


def kernel(logits):
    return _tc_argmax(logits).reshape(NROWS)


# speedup vs baseline: 1.1203x; 1.1093x over previous
import numpy as np
import jax
import jax.numpy as jnp
from jax import lax
from jax.experimental import pallas as pl
from jax.experimental.pallas import tpu as pltpu

NROWS = 64
NCOLS = 1_000_000
BLK = 16_384
GRID = (NCOLS + BLK - 1) // BLK
GROUPS = BLK // 128

_K0 = np.uint32(0)
_K1 = np.uint32(42)
_KS2 = np.uint32((0 ^ 42) ^ 0x1BD11BDA)
_ROT_A = (13, 15, 26, 6)
_ROT_B = (17, 29, 16, 24)
_MIN = np.float32(1e-7)
_SCALE = np.float32(np.float32(1.0 - 1e-7) - np.float32(1e-7))


def _rotl(x, r):
    return (x << np.uint32(r)) | (x >> np.uint32(32 - r))


def _rounds(x0, x1, rs):
    for r in rs:
        x0 = x0 + x1
        x1 = _rotl(x1, r)
        x1 = x1 ^ x0
    return x0, x1


def _threefry_bits(flat_u32):
    x0 = jnp.full(flat_u32.shape, _K0, jnp.uint32)
    x1 = flat_u32 + _K1
    x0, x1 = _rounds(x0, x1, _ROT_A)
    x0 = x0 + _K1
    x1 = x1 + (_KS2 + np.uint32(1))
    x0, x1 = _rounds(x0, x1, _ROT_B)
    x0 = x0 + _KS2
    x1 = x1 + (_K0 + np.uint32(2))
    x0, x1 = _rounds(x0, x1, _ROT_A)
    x0 = x0 + _K0
    x1 = x1 + (_K1 + np.uint32(3))
    x0, x1 = _rounds(x0, x1, _ROT_B)
    x0 = x0 + _K1
    x1 = x1 + (_KS2 + np.uint32(4))
    x0, x1 = _rounds(x0, x1, _ROT_A)
    x0 = x0 + _KS2
    x1 = x1 + (_K0 + np.uint32(5))
    return x0 ^ x1


def _gumbel(flat_u32):
    bits = _threefry_bits(flat_u32)
    fb = (bits >> np.uint32(9)) | np.uint32(0x3F800000)
    fl = lax.bitcast_convert_type(fb, jnp.float32) - jnp.float32(1.0)
    u = jnp.maximum(_MIN, fl * _SCALE + _MIN)
    return -jnp.log(-jnp.log(u))


def _tc_body(lref, outref, rmax, ridx):
    i = pl.program_id(0)

    @pl.when(i == 0)
    def _():
        rmax[...] = jnp.full((NROWS, 128), -jnp.inf, jnp.float32)
        ridx[...] = jnp.zeros((NROWS, 128), jnp.int32)

    base = i * BLK
    lane = lax.broadcasted_iota(jnp.int32, (NROWS, 128), 1)
    rowc = lax.broadcasted_iota(jnp.int32, (NROWS, 128), 0) * NCOLS

    rm = rmax[...]
    ri = ridx[...]

    UNROLL = 4

    def group(g, car):
        rm, ri = car
        for sub in range(UNROLL):
            off = (g * UNROLL + sub) * 128
            col = (base + off) + lane
            flat = (rowc + col).astype(jnp.uint32)
            gum = _gumbel(flat)
            v = lref[:, pl.ds(off, 128)] + gum
            valid = col < NCOLS
            v = jnp.where(valid, v, -jnp.inf)
            m = v > rm
            rm = jnp.where(m, v, rm)
            ri = jnp.where(m, col, ri)
        return rm, ri

    rm, ri = lax.fori_loop(0, GROUPS // UNROLL, group, (rm, ri))
    rmax[...] = rm
    ridx[...] = ri

    @pl.when(i == GRID - 1)
    def _():
        mval = jnp.max(rm, axis=1, keepdims=True)
        cand = jnp.where(rm == mval, ri, jnp.int32(2**31 - 1))
        outref[...] = jnp.min(cand, axis=1, keepdims=True)


_tc_argmax = pl.pallas_call(
    _tc_body,
    grid=(GRID,),
    in_specs=[
        pl.BlockSpec((NROWS, BLK), lambda i: (0, i)),
    ],
    out_specs=pl.BlockSpec((NROWS, 1), lambda i: (0, 0)),
    out_shape=jax.ShapeDtypeStruct((NROWS, 1), jnp.int32),
    scratch_shapes=[
        pltpu.VMEM((NROWS, 128), jnp.float32),
        pltpu.VMEM((NROWS, 128), jnp.int32),
    ],
)


def kernel(logits):
    return _tc_argmax(logits).reshape(NROWS)


# speedup vs baseline: 1.1298x; 1.0085x over previous
import numpy as np
import jax
import jax.numpy as jnp
from jax import lax
from jax.experimental import pallas as pl
from jax.experimental.pallas import tpu as pltpu

NROWS = 64
NCOLS = 1_000_000
BLK = 16_384
GRID = (NCOLS + BLK - 1) // BLK
GROUPS = BLK // 128

_K0 = np.uint32(0)
_K1 = np.uint32(42)
_KS2 = np.uint32((0 ^ 42) ^ 0x1BD11BDA)
_ROT_A = (13, 15, 26, 6)
_ROT_B = (17, 29, 16, 24)
_MIN = np.float32(1e-7)
_SCALE = np.float32(np.float32(1.0 - 1e-7) - np.float32(1e-7))


def _rotl(x, r):
    return (x << np.uint32(r)) | (x >> np.uint32(32 - r))


def _rounds(x0, x1, rs):
    for r in rs:
        x0 = x0 + x1
        x1 = _rotl(x1, r)
        x1 = x1 ^ x0
    return x0, x1


def _threefry_bits(flat_u32):
    x0 = jnp.full(flat_u32.shape, _K0, jnp.uint32)
    x1 = flat_u32 + _K1
    x0, x1 = _rounds(x0, x1, _ROT_A)
    x0 = x0 + _K1
    x1 = x1 + (_KS2 + np.uint32(1))
    x0, x1 = _rounds(x0, x1, _ROT_B)
    x0 = x0 + _KS2
    x1 = x1 + (_K0 + np.uint32(2))
    x0, x1 = _rounds(x0, x1, _ROT_A)
    x0 = x0 + _K0
    x1 = x1 + (_K1 + np.uint32(3))
    x0, x1 = _rounds(x0, x1, _ROT_B)
    x0 = x0 + _K1
    x1 = x1 + (_KS2 + np.uint32(4))
    x0, x1 = _rounds(x0, x1, _ROT_A)
    x0 = x0 + _KS2
    x1 = x1 + (_K0 + np.uint32(5))
    return x0 ^ x1


def _gumbel(flat_u32):
    bits = _threefry_bits(flat_u32)
    fb = (bits >> np.uint32(9)) | np.uint32(0x3F800000)
    fl = lax.bitcast_convert_type(fb, jnp.float32) - jnp.float32(1.0)
    u = jnp.maximum(_MIN, fl * _SCALE + _MIN)
    return -jnp.log(-jnp.log(u))


def _tc_body(lref, outref, rmax, ridx):
    i = pl.program_id(0)

    @pl.when(i == 0)
    def _():
        rmax[...] = jnp.full((NROWS, 128), -jnp.inf, jnp.float32)
        ridx[...] = jnp.zeros((NROWS, 128), jnp.int32)

    base = i * BLK
    lane = lax.broadcasted_iota(jnp.int32, (NROWS, 128), 1)
    rowc = lax.broadcasted_iota(jnp.int32, (NROWS, 128), 0) * NCOLS

    rm = rmax[...]
    ri = ridx[...]

    UNROLL = 8

    def group(g, car):
        rm, ri = car
        for sub in range(UNROLL):
            off = (g * UNROLL + sub) * 128
            col = (base + off) + lane
            flat = (rowc + col).astype(jnp.uint32)
            gum = _gumbel(flat)
            v = lref[:, pl.ds(off, 128)] + gum
            valid = col < NCOLS
            v = jnp.where(valid, v, -jnp.inf)
            m = v > rm
            rm = jnp.where(m, v, rm)
            ri = jnp.where(m, col, ri)
        return rm, ri

    rm, ri = lax.fori_loop(0, GROUPS // UNROLL, group, (rm, ri))
    rmax[...] = rm
    ridx[...] = ri

    @pl.when(i == GRID - 1)
    def _():
        mval = jnp.max(rm, axis=1, keepdims=True)
        cand = jnp.where(rm == mval, ri, jnp.int32(2**31 - 1))
        outref[...] = jnp.min(cand, axis=1, keepdims=True)


_tc_argmax = pl.pallas_call(
    _tc_body,
    grid=(GRID,),
    in_specs=[
        pl.BlockSpec((NROWS, BLK), lambda i: (0, i)),
    ],
    out_specs=pl.BlockSpec((NROWS, 1), lambda i: (0, 0)),
    out_shape=jax.ShapeDtypeStruct((NROWS, 1), jnp.int32),
    scratch_shapes=[
        pltpu.VMEM((NROWS, 128), jnp.float32),
        pltpu.VMEM((NROWS, 128), jnp.int32),
    ],
)


def kernel(logits):
    return _tc_argmax(logits).reshape(NROWS)
